# Initial kernel scaffold; baseline (speedup 1.0000x reference)
import jax
import jax.numpy as jnp
from jax.experimental import pallas as pl


def kernel(x, edge_index, batch, mpnn_W, mpnn_b, mpnn_Wu, mpnn_bu, g1_Wl, g1_Wr, g1_att, g1_Wres, g1_bias, g2_Wl, g2_Wr, g2_att, g2_Wres, g2_bias, g3_Wl, g3_Wr, g3_att, g3_Wres, g3_bias, sage_Wl, sage_bl, sage_Wr, out_W, out_b):
    raise NotImplementedError("write your pallas kernel here")


# speedup vs baseline: 25.7756x; 25.7756x over previous
import functools

import jax
import jax.numpy as jnp
from jax import lax
from jax.experimental import pallas as pl
from jax.experimental.pallas import tpu as pltpu
from jax.experimental.pallas import tpu_sc as plsc

F32 = jnp.float32
HID, H, C, NGRAPH = 16, 8, 16, 16
NC, NS = 2, 16
NW = NC * NS
CHUNK = 128
N_PAD = 10112
RPT = N_PAD // NS
BR = 2528
GRID = N_PAD // BR

_mesh = functools.partial(
    plsc.VectorSubcoreMesh, core_axis_name="c", subcore_axis_name="s")

_PERM_DNUMS = lax.GatherDimensionNumbers(
    offset_dims=(), collapsed_slice_dims=(0,), start_index_map=(0,))


def _lane_shuffle(v, idx):
    return lax.gather(v, idx[:, None], _PERM_DNUMS, (1,),
                      mode=lax.GatherScatterMode.PROMISE_IN_BOUNDS)


def _lane_allsum(v, perms):
    for p in perms:
        v = v + _lane_shuffle(v, p)
    return v


def _mm(a, b):
    return jnp.dot(a, b, precision=jax.lax.Precision.HIGHEST,
                   preferred_element_type=F32)


def _make_gather_scatter(width, n_chunks):
    cpt = n_chunks // NW

    @functools.partial(
        pl.kernel,
        out_type=jax.ShapeDtypeStruct((NC, N_PAD, width), F32),
        mesh=_mesh(),
        scratch_types=[
            pltpu.VMEM((CHUNK,), jnp.int32),
            pltpu.VMEM((CHUNK,), jnp.int32),
            pltpu.VMEM((CHUNK, width), F32),
            pltpu.VMEM_SHARED((N_PAD, width), F32),
            pltpu.SemaphoreType.DMA,
        ],
    )
    def k(table, srcl, dstl, zrows, out, idx_s, idx_d, rows, accum, sem):
        cid = lax.axis_index("c")
        sid = lax.axis_index("s")
        wid = cid * NS + sid
        r0 = sid * RPT
        pltpu.sync_copy(zrows.at[pl.ds(r0, RPT)], accum.at[pl.ds(r0, RPT)])
        plsc.subcore_barrier()

        def chunk(i, carry):
            base = (wid * cpt + i) * CHUNK
            pltpu.sync_copy(srcl.at[pl.ds(base, CHUNK)], idx_s)
            pltpu.sync_copy(dstl.at[pl.ds(base, CHUNK)], idx_d)
            pltpu.async_copy(table.at[idx_s], rows, sem).wait()
            pltpu.sync_copy(rows, accum.at[idx_d], add=True)
            return carry

        lax.fori_loop(0, cpt, chunk, 0)
        plsc.subcore_barrier()
        pltpu.sync_copy(accum.at[pl.ds(r0, RPT)],
                        out.at[cid, pl.ds(r0, RPT)])

    return k


def _make_gat_pass(n_chunks):
    cpt = n_chunks // NS
    HH = H // NC

    @functools.partial(
        pl.kernel,
        out_type=jax.ShapeDtypeStruct((NC, N_PAD, H * C), F32),
        mesh=_mesh(),
        scratch_types=[
            pltpu.VMEM((CHUNK,), jnp.int32),
            pltpu.VMEM((CHUNK,), jnp.int32),
            pltpu.VMEM((CHUNK, H * C), F32),
            pltpu.VMEM((CHUNK, H * C), F32),
            pltpu.VMEM((H, C), F32),
            pltpu.VMEM_SHARED((N_PAD, H * C), F32),
            pltpu.SemaphoreType.DMA,
            pltpu.SemaphoreType.DMA,
        ],
    )
    def k(xl, xr, srcl, dstl, att, zrows, out,
          idx_s, idx_d, rows_l, rows_r, att_v, accum, sem1, sem2):
        cid = lax.axis_index("c")
        sid = lax.axis_index("s")
        r0 = sid * RPT
        pltpu.sync_copy(att, att_v)
        pltpu.sync_copy(zrows.at[pl.ds(r0, RPT)], accum.at[pl.ds(r0, RPT)])
        plsc.subcore_barrier()

        lane = jax.lax.iota(jnp.int32, 16)
        perms = [jnp.bitwise_xor(lane, k) for k in (8, 4, 2, 1)]
        atts = [att_v[h, :] for h in range(H)]

        def edge_body(h_base):
            def edge(e, ecarry):
                for hk in range(HH):
                    off = (h_base + hk) * C
                    a = rows_l[e, pl.ds(off, 16)]
                    b = rows_r[e, pl.ds(off, 16)]
                    z = a + b
                    z = jnp.maximum(z, 0.2 * z)
                    w = jnp.exp(_lane_allsum(z * atts[h_base + hk], perms))
                    rows_l[e, pl.ds(16 * hk, 16)] = a * w
                    rows_l[e, pl.ds(64 + 16 * hk, 16)] = w
                return ecarry
            return edge

        def chunk(i, carry):
            base = (sid * cpt + i) * CHUNK
            pltpu.sync_copy(srcl.at[pl.ds(base, CHUNK)], idx_s)
            pltpu.sync_copy(dstl.at[pl.ds(base, CHUNK)], idx_d)
            cp1 = pltpu.async_copy(xl.at[idx_s], rows_l, sem1)
            cp2 = pltpu.async_copy(xr.at[idx_d], rows_r, sem2)
            cp1.wait()
            cp2.wait()

            @pl.when(cid == 0)
            def _():
                lax.fori_loop(0, CHUNK, edge_body(0), 0)

            @pl.when(cid != 0)
            def _():
                lax.fori_loop(0, CHUNK, edge_body(HH), 0)

            pltpu.sync_copy(rows_l, accum.at[idx_d], add=True)
            return carry

        lax.fori_loop(0, cpt, chunk, 0)
        plsc.subcore_barrier()
        pltpu.sync_copy(accum.at[pl.ds(r0, RPT)],
                        out.at[cid, pl.ds(r0, RPT)])

    return k


def _mpnn_pre(x_ref, w_ref, b_ref, o_ref):
    h = _mm(x_ref[...], w_ref[...]) + b_ref[...]
    n = h.shape[0]
    o_ref[...] = jnp.concatenate(
        [h, jnp.ones((n, 1), F32), jnp.zeros((n, 111), F32)], axis=1)


def _mpnn_update(part_ref, hext_ref, wu_ref, bu_ref, wl_ref, wr_ref,
                 h0_ref, xl_ref, xr_ref, deg_ref):
    acc = part_ref[0] + part_ref[1]
    cat = jnp.concatenate([hext_ref[:, :16], acc[:, :16]], axis=1)
    u = _mm(cat, wu_ref[...]) + bu_ref[...]
    u = jnp.maximum(u, 0.1 * u)
    h0 = jnp.maximum(u, 0.0)
    h0_ref[...] = h0
    xl_ref[...] = _mm(h0, wl_ref[...])
    xr_ref[...] = _mm(h0, wr_ref[...])
    deg_ref[...] = acc[:, 16:17]


def _elu(x):
    return jnp.where(x > 0, x, jnp.exp(jnp.minimum(x, 0.0)) - 1.0)


def _make_gat_finalize(act, emit_next):
    def _gat(part_ref):
        p0, p1 = part_ref[0], part_ref[1]
        return jnp.concatenate(
            [p0[:, :64] / (p0[:, 64:] + 1e-16),
             p1[:, :64] / (p1[:, 64:] + 1e-16)], axis=1)

    if emit_next:
        def body(part_ref, hin_ref, wres_ref, bias_ref, wl_ref, wr_ref,
                 hout_ref, xl_ref, xr_ref):
            h = act(_gat(part_ref) + _mm(hin_ref[...], wres_ref[...])
                    + bias_ref[...])
            hout_ref[...] = h
            xl_ref[...] = _mm(h, wl_ref[...])
            xr_ref[...] = _mm(h, wr_ref[...])
    else:
        def body(part_ref, hin_ref, wres_ref, bias_ref, hout_ref):
            hout_ref[...] = act(
                _gat(part_ref) + _mm(hin_ref[...], wres_ref[...])
                + bias_ref[...])
    return body


def _sage_pool(h3_ref, part_ref, deg_ref, batch_ref, wl_ref, bl_ref,
               wr_ref, ow_ref, ob_ref, out_ref, sums, cnt):
    i = pl.program_id(0)
    nb = part_ref[0] + part_ref[1]
    mean = nb / jnp.maximum(deg_ref[...], 1.0)
    h4 = _mm(mean, wl_ref[...]) + bl_ref[...] + _mm(h3_ref[...], wr_ref[...])
    h4 = jnp.maximum(h4, 0.0)
    n = h4.shape[0]
    onehot = (batch_ref[...] ==
              jax.lax.broadcasted_iota(jnp.int32, (n, NGRAPH), 1)).astype(F32)
    s_c = jax.lax.dot_general(onehot, h4, (((0,), (0,)), ((), ())),
                              precision=jax.lax.Precision.HIGHEST,
                              preferred_element_type=F32)
    c_c = jax.lax.dot_general(onehot, jnp.ones((n, 1), F32),
                              (((0,), (0,)), ((), ())),
                              precision=jax.lax.Precision.HIGHEST,
                              preferred_element_type=F32)

    @pl.when(i == 0)
    def _():
        sums[...] = s_c
        cnt[...] = c_c

    @pl.when(i > 0)
    def _():
        sums[...] += s_c
        cnt[...] += c_c

    @pl.when(i == GRID - 1)
    def _():
        pooled = sums[...] / jnp.maximum(cnt[...], 1.0)
        logits = _mm(pooled, ow_ref[...]) + ob_ref[...]
        m = jnp.max(logits, axis=1, keepdims=True)
        lse = jnp.log(jnp.sum(jnp.exp(logits - m), axis=1, keepdims=True)) + m
        out_ref[...] = logits - lse


def _full(shape):
    return pl.BlockSpec(shape, lambda i: tuple(0 for _ in shape))


def _rows(width):
    return pl.BlockSpec((BR, width), lambda i: (i, 0))


def _rows3(lead, width):
    return pl.BlockSpec((lead, BR, width), lambda i: (0, i, 0))


def kernel(x, edge_index, batch, mpnn_W, mpnn_b, mpnn_Wu, mpnn_bu,
           g1_Wl, g1_Wr, g1_att, g1_Wres, g1_bias,
           g2_Wl, g2_Wr, g2_att, g2_Wres, g2_bias,
           g3_Wl, g3_Wr, g3_att, g3_Wres, g3_bias,
           sage_Wl, sage_bl, sage_Wr, out_W, out_b):
    n = x.shape[0]
    e = edge_index.shape[1]
    src, dst = edge_index[0], edge_index[1]

    unit = CHUNK * NW
    e_pad = ((e + unit - 1) // unit) * unit
    pe = e_pad - e
    src_e = jnp.concatenate([src, jnp.zeros((pe,), jnp.int32)])
    dst_e = jnp.concatenate([dst, jnp.full((pe,), n, jnp.int32)])

    esl = e + n
    esl_pad = ((esl + unit - 1) // unit) * unit
    pg = esl_pad - esl
    loop = jnp.arange(n, dtype=jnp.int32)
    gat_src = jnp.concatenate([src, loop, jnp.zeros((pg,), jnp.int32)])
    gat_dst = jnp.concatenate([jnp.where(src != dst, dst, n), loop,
                               jnp.full((pg,), n, jnp.int32)])

    x_pad = jnp.pad(x, ((0, N_PAD - n), (0, 0)))
    batch_col = jnp.pad(batch, (0, N_PAD - n),
                        constant_values=NGRAPH).reshape(N_PAD, 1)
    z128 = jnp.zeros((N_PAD, H * C), F32)
    b2 = lambda v: v.reshape(1, -1)

    seg128 = _make_gather_scatter(H * C, e_pad // CHUNK)
    gat_pass = _make_gat_pass(esl_pad // CHUNK)

    h_ext = pl.pallas_call(
        _mpnn_pre,
        out_shape=jax.ShapeDtypeStruct((N_PAD, H * C), F32),
    )(x_pad, mpnn_W, b2(mpnn_b))

    mpnn_part = seg128(h_ext, src_e, dst_e, z128)

    h0, xl1, xr1, deg = pl.pallas_call(
        _mpnn_update,
        grid=(GRID,),
        in_specs=[_rows3(2, 128), _rows(128), _full((32, 16)), _full((1, 16)),
                  _full((16, 128)), _full((16, 128))],
        out_specs=[_rows(16), _rows(128), _rows(128), _rows(1)],
        out_shape=[jax.ShapeDtypeStruct((N_PAD, 16), F32),
                   jax.ShapeDtypeStruct((N_PAD, 128), F32),
                   jax.ShapeDtypeStruct((N_PAD, 128), F32),
                   jax.ShapeDtypeStruct((N_PAD, 1), F32)],
    )(mpnn_part, h_ext, mpnn_Wu, b2(mpnn_bu), g1_Wl, g1_Wr)

    h_in = h0
    xl, xr = xl1, xr1
    layers = [
        (g1_att, g1_Wres, g1_bias, _elu, g2_Wl, g2_Wr),
        (g2_att, g2_Wres, g2_bias,
         lambda v: jnp.maximum(v, 0.01 * v), g3_Wl, g3_Wr),
        (g3_att, g3_Wres, g3_bias, _elu, None, None),
    ]
    for li, (att, wres, bias, act, wln, wrn) in enumerate(layers):
        part = gat_pass(xl, xr, gat_src, gat_dst, att.reshape(H, C), z128)
        emit_next = wln is not None
        din = h_in.shape[1]
        ins = [part, h_in, wres, b2(bias)]
        specs = [_rows3(2, H * C), _rows(din), _full((din, 128)),
                 _full((1, 128))]
        nout = 1
        if emit_next:
            ins += [wln, wrn]
            specs += [_full((128, 128)), _full((128, 128))]
            nout = 3
        res = pl.pallas_call(
            _make_gat_finalize(act, emit_next),
            grid=(GRID,),
            in_specs=specs,
            out_specs=[_rows(128)] * nout,
            out_shape=[jax.ShapeDtypeStruct((N_PAD, 128), F32)] * nout,
        )(*ins)
        if emit_next:
            h_in, xl, xr = res
        else:
            h_in = res[0]
    h3 = h_in

    sage_part = seg128(h3, src_e, dst_e, z128)

    out = pl.pallas_call(
        _sage_pool,
        grid=(GRID,),
        in_specs=[_rows(128), _rows3(2, 128), _rows(1),
                  pl.BlockSpec((BR, 1), lambda i: (i, 0)),
                  _full((128, 16)), _full((1, 16)), _full((128, 16)),
                  _full((16, 2)), _full((1, 2))],
        out_specs=_full((NGRAPH, 2)),
        out_shape=jax.ShapeDtypeStruct((NGRAPH, 2), F32),
        scratch_shapes=[pltpu.VMEM((NGRAPH, NGRAPH), F32),
                        pltpu.VMEM((NGRAPH, 1), F32)],
    )(h3, sage_part, deg, batch_col, sage_Wl, b2(sage_bl), sage_Wr,
      out_W, b2(out_b))
    return out
